# baseline (speedup 1.0000x reference)
import jax
from jax import lax
import jax.numpy as jnp
from jax.experimental import pallas as pl
from jax.experimental.pallas import tpu as pltpu
from jax.experimental.pallas import tpu_sc as plsc

_W = 1024
_NW = 32
_NSUB = 16


def kernel(emb_ids, table):
    bsz, seq = emb_ids.shape
    num_rows, dim = table.shape
    n = bsz * seq
    chunks = n // _W
    per_w = chunks // _NW
    steps_per_seq = bsz // _W
    rows_per_sub = num_rows // _NSUB
    a_chunks = (rows_per_sub + _W - 1) // _W
    a_chunks += a_chunks % 2
    idx = emb_ids.T.reshape(1, n)
    flat = table.T.reshape(num_rows * dim)

    mesh = plsc.VectorSubcoreMesh(core_axis_name="core", subcore_axis_name="subcore")

    @pl.kernel(
        out_type=[
            jax.ShapeDtypeStruct((seq, dim, bsz), table.dtype),
            jax.ShapeDtypeStruct((2, num_rows, dim), table.dtype),
        ],
        mesh=mesh,
        scratch_types=[
            pltpu.VMEM((per_w * _W,), jnp.int32),
            pltpu.VMEM((2, _W, dim), table.dtype),
            pltpu.VMEM((2, dim, _W), table.dtype),
            pltpu.SemaphoreType.DMA,
            pltpu.SemaphoreType.DMA((2,)),
            pltpu.SemaphoreType.DMA((2,)),
            pltpu.SemaphoreType.DMA((2,)),
        ],
        compiler_params=pltpu.CompilerParams(
            use_tc_tiling_on_sc=False, needs_layout_passes=False
        ),
    )
    def _gather_kernel(x_hbm, i_hbm, o_hbm, scr_hbm, ids_v, g_v, t_v, s_i, s_g, s_o, s_a):
        cid = lax.axis_index("core")
        sid = lax.axis_index("subcore")
        wid = sid * 2 + cid
        lane = lax.iota(jnp.int32, 16)
        my_scr = scr_hbm.at[cid]

        a_base = sid * num_rows // _NSUB // 8 * 8
        a_end = (sid + 1) * num_rows // _NSUB // 8 * 8

        def a_off(j):
            return a_base + jnp.minimum(j * _W, a_end - a_base - _W)

        def a_start(j, b):
            off = a_off(j)
            for c in range(dim):
                pltpu.async_copy(
                    x_hbm.at[pl.ds(c * num_rows + off, _W)],
                    t_v.at[b, c],
                    s_a.at[b],
                )

        def a_half(j, b):
            off = a_off(j)
            for c in range(dim):
                pltpu.make_async_copy(
                    x_hbm.at[pl.ds(c * num_rows + off, _W)],
                    t_v.at[b, c],
                    s_a.at[b],
                ).wait()

            @pl.when(j + 1 < a_chunks)
            def _():
                a_start(j + 1, 1 - b)

            @pl.loop(0, _W // 16)
            def _(kk):
                for jj in range(16):
                    t = kk * 16 + jj
                    g_v[b, t, :] = plsc.load_gather(
                        t_v.at[b], [lane, jnp.full((16,), t, jnp.int32)]
                    )

            pltpu.async_copy(
                g_v.at[b], my_scr.at[pl.ds(off, _W), :], s_o.at[b]
            ).wait()

        a_start(0, 0)

        @pl.loop(0, a_chunks, step=2)
        def _(j):
            a_half(j, 0)
            a_half(j + 1, 1)

        plsc.subcore_barrier()

        base = wid * per_w
        pltpu.async_copy(
            i_hbm.at[0, pl.ds(base * _W, per_w * _W)], ids_v, s_i
        ).wait()

        def gather_start(k):
            b = k % 2
            return pltpu.async_copy(
                my_scr.at[ids_v.at[pl.ds(k * _W, _W)]], g_v.at[b], s_g.at[b]
            )

        def transpose(b):
            @pl.loop(0, _W // 16)
            def _(kk):
                rows = kk * 16 + lane
                for c in range(dim):
                    t_v[b, c, pl.ds(kk * 16, 16)] = plsc.load_gather(
                        g_v.at[b], [rows, jnp.full((16,), c, jnp.int32)]
                    )

        def out_start(k):
            b = k % 2
            kid = base + k
            l = kid // steps_per_seq
            b0 = (kid % steps_per_seq) * _W
            return pltpu.async_copy(
                t_v.at[b], o_hbm.at[l, :, pl.ds(b0, _W)], s_o.at[b]
            )

        g_h = {0: gather_start(0)}
        o_h = {}
        for k in range(per_w):
            if k + 1 < per_w:
                g_h[(k + 1) % 2] = gather_start(k + 1)
            g_h[k % 2].wait()
            if k >= 2:
                o_h[k % 2].wait()
            transpose(k % 2)
            o_h[k % 2] = out_start(k)
        o_h[(per_w - 1) % 2].wait()
        o_h[(per_w - 2) % 2].wait()

    out, _ = _gather_kernel(flat, idx)
    return out.transpose(2, 0, 1)


# speedup vs baseline: 3.2956x; 3.2956x over previous
import jax
from jax import lax
import jax.numpy as jnp
from jax.experimental import pallas as pl
from jax.experimental.pallas import tpu as pltpu
from jax.experimental.pallas import tpu_sc as plsc

_W = 1024
_NW = 32


def kernel(emb_ids, table):
    bsz, seq = emb_ids.shape
    num_rows, dim = table.shape
    n = bsz * seq
    chunks = n // _W
    per_w = chunks // _NW
    steps_per_seq = bsz // _W
    idx = emb_ids.T.reshape(1, n)

    mesh = plsc.VectorSubcoreMesh(core_axis_name="core", subcore_axis_name="subcore")

    @pl.kernel(
        out_type=jax.ShapeDtypeStruct((seq, dim, bsz), table.dtype),
        mesh=mesh,
        scratch_types=[
            pltpu.VMEM((per_w * _W,), jnp.int32),
            pltpu.VMEM((2, _W, dim), table.dtype),
            pltpu.VMEM((2, dim, _W), table.dtype),
            pltpu.SemaphoreType.DMA,
            pltpu.SemaphoreType.DMA((2,)),
            pltpu.SemaphoreType.DMA((2,)),
        ],
        compiler_params=pltpu.CompilerParams(
            use_tc_tiling_on_sc=False, needs_layout_passes=False
        ),
    )
    def _gather_kernel(x_hbm, i_hbm, o_hbm, ids_v, g_v, t_v, s_i, s_g, s_o):
        wid = lax.axis_index("subcore") * 2 + lax.axis_index("core")
        base = wid * per_w
        pltpu.async_copy(
            i_hbm.at[0, pl.ds(base * _W, per_w * _W)], ids_v, s_i
        ).wait()
        lane = lax.iota(jnp.int32, 16)

        def gather_start(k):
            b = k % 2
            return pltpu.async_copy(
                x_hbm.at[ids_v.at[pl.ds(k * _W, _W)]], g_v.at[b], s_g.at[b]
            )

        def transpose(b):
            @pl.loop(0, _W // 16)
            def _(kk):
                rows = kk * 16 + lane
                for c in range(dim):
                    t_v[b, c, pl.ds(kk * 16, 16)] = plsc.load_gather(
                        g_v.at[b], [rows, jnp.full((16,), c, jnp.int32)]
                    )

        def out_start(k):
            b = k % 2
            cid = base + k
            l = cid // steps_per_seq
            b0 = (cid % steps_per_seq) * _W
            return pltpu.async_copy(
                t_v.at[b], o_hbm.at[l, :, pl.ds(b0, _W)], s_o.at[b]
            )

        g_h = {0: gather_start(0)}
        o_h = {}
        for k in range(per_w):
            if k + 1 < per_w:
                g_h[(k + 1) % 2] = gather_start(k + 1)
            g_h[k % 2].wait()
            if k >= 2:
                o_h[k % 2].wait()
            transpose(k % 2)
            o_h[k % 2] = out_start(k)
        o_h[(per_w - 1) % 2].wait()
        o_h[(per_w - 2) % 2].wait()

    out = _gather_kernel(table, idx)
    return out.transpose(2, 0, 1)
